# baseline (speedup 1.0000x reference)
import jax
import jax.numpy as jnp
from jax import lax
from jax.experimental import pallas as pl
from jax.experimental.pallas import tpu as pltpu


def _roll(x, sh, axis):
    return pltpu.roll(x, sh, axis)


def _lovasz_body(p_ref, t_ref, o_ref):
    p = p_ref[0]
    t = t_ref[0]
    R, C = p.shape
    N = R * C
    LOGN = N.bit_length() - 1

    row = lax.broadcasted_iota(jnp.int32, (R, C), 0)
    lane = lax.broadcasted_iota(jnp.int32, (R, C), 1)

    signs = 2.0 * t - 1.0
    e = 1.0 - p * signs
    bits = lax.bitcast_convert_type(e, jnp.int32)
    bits = (bits & jnp.int32(-2)) | t.astype(jnp.int32)
    y = bits ^ ((bits >> 31) & jnp.int32(0x7FFFFFFF))
    z = ~y

    def bit0(s):
        if s >= C:
            return (row & (s // C)) == 0
        return (lane & s) == 0

    idx = row * C + lane
    z = z ^ -((idx >> 1) & 1)
    for k in range(1, LOGN + 1):
        for j in range(k - 1, -1, -1):
            s = 1 << j
            bs = bit0(s)
            if s >= C:
                u = s // C
                w = _roll(z, u, 0)
                m1 = jnp.minimum(z, w)
                m2 = jnp.maximum(z, w)
                mu = _roll(m1, R - u, 0)
            else:
                w = _roll(z, s, 1)
                m1 = jnp.minimum(z, w)
                m2 = jnp.maximum(z, w)
                mu = _roll(m1, C - s, 1)
            z = jnp.where(bs, mu, m2)
        if k < LOGN:
            z = z ^ -(((idx >> k) ^ (idx >> (k + 1))) & 1)

    y_s = ~z
    bits_s = y_s ^ ((y_s >> 31) & jnp.int32(0x7FFFFFFF))
    t_s = (bits_s & 1).astype(jnp.float32)
    e_s = lax.bitcast_convert_type(bits_s, jnp.float32)

    ia = lax.broadcasted_iota(jnp.int32, (C, C), 0)
    ib = lax.broadcasted_iota(jnp.int32, (C, C), 1)
    tri = (ia <= ib).astype(jnp.float32)
    lanecum = jnp.dot(t_s, tri, preferred_element_type=jnp.float32)
    rowsum = lanecum[:, C - 1:C]
    acc = rowsum
    sh = 1
    while sh < R:
        shifted = jnp.concatenate([jnp.zeros((sh, 1), jnp.float32), acc[:-sh, :]], axis=0)
        acc = acc + shifted
        sh *= 2
    cum_t = lanecum + (acc - rowsum)

    gts = jnp.sum(t_s)
    cnt = (row * C + lane + 1).astype(jnp.float32)
    cum1 = cnt - cum_t
    inter = gts - cum_t
    union = gts + cum1
    jacc = 1.0 - inter / jnp.maximum(union, 1e-6)
    wrapped = _roll(jacc, 1, 1)
    lastcol_dn = jnp.concatenate(
        [jnp.zeros((1, 1), jnp.float32), jacc[:-1, C - 1:C]], axis=0)
    prev = jnp.where(lane == 0, lastcol_dn, wrapped)
    grad = jacc - prev
    loss = jnp.sum(jnp.maximum(e_s, 0.0) * grad)
    o_ref[0, 0, :] = jnp.broadcast_to(loss, (C,))


def _run(pred, target, interpret=False):
    B = pred.shape[0]
    C = 128
    R = (pred.shape[1] * pred.shape[2]) // C
    p = pred.reshape(B, R, C)
    t = target.reshape(B, R, C)
    losses = pl.pallas_call(
        _lovasz_body,
        grid=(B,),
        in_specs=[
            pl.BlockSpec((1, R, C), lambda i: (i, 0, 0)),
            pl.BlockSpec((1, R, C), lambda i: (i, 0, 0)),
        ],
        out_specs=pl.BlockSpec((1, 1, C), lambda i: (i, 0, 0)),
        out_shape=jax.ShapeDtypeStruct((B, 1, C), jnp.float32),
        interpret=interpret,
    )(p, t)
    total = jnp.sum(losses[:, 0, 0]) / B
    return jnp.where(jnp.isfinite(total), total, jnp.zeros((), jnp.float32))


def kernel(pred, target):
    return _run(pred, target)


# speedup vs baseline: 1.3167x; 1.3167x over previous
import jax
import jax.numpy as jnp
from jax import lax
from jax.experimental import pallas as pl


def _roll(x, sh, axis):
    if axis == 0:
        return jnp.concatenate([x[-sh:, :], x[:-sh, :]], axis=0)
    return jnp.concatenate([x[:, -sh:], x[:, :-sh]], axis=1)


def _lovasz_body(p_ref, t_ref, o_ref):
    p = p_ref[0]
    t = t_ref[0]
    R, C = p.shape
    N = R * C
    LOGN = N.bit_length() - 1

    row = lax.broadcasted_iota(jnp.int32, (R, C), 0)
    lane = lax.broadcasted_iota(jnp.int32, (R, C), 1)

    signs = 2.0 * t - 1.0
    e = 1.0 - p * signs
    bits = lax.bitcast_convert_type(e, jnp.int32)
    bits = (bits & jnp.int32(-2)) | t.astype(jnp.int32)
    y = bits ^ ((bits >> 31) & jnp.int32(0x7FFFFFFF))
    z = ~y

    idx = lane * R + row

    def bit0(s):
        if s >= R:
            return (lane & (s // R)) == 0
        return (row & s) == 0

    z = z ^ -((idx >> 1) & 1)
    for k in range(1, LOGN + 1):
        for j in range(k - 1, -1, -1):
            s = 1 << j
            bs = bit0(s)
            if s >= R:
                u = s // R
                w = _roll(z, u, 1)
                m1 = jnp.minimum(z, w)
                m2 = jnp.maximum(z, w)
                mu = _roll(m1, C - u, 1)
            else:
                w = _roll(z, s, 0)
                m1 = jnp.minimum(z, w)
                m2 = jnp.maximum(z, w)
                mu = _roll(m1, R - s, 0)
            z = jnp.where(bs, mu, m2)
        if k < LOGN:
            z = z ^ -(((idx >> k) ^ (idx >> (k + 1))) & 1)

    y_s = ~z
    bits_s = y_s ^ ((y_s >> 31) & jnp.int32(0x7FFFFFFF))
    t_s = (bits_s & 1).astype(jnp.float32)
    e_s = lax.bitcast_convert_type(bits_s, jnp.float32)

    acc = t_s
    sh = 1
    while sh < R:
        shifted = jnp.concatenate(
            [jnp.zeros((sh, C), jnp.float32), acc[:-sh, :]], axis=0)
        acc = acc + shifted
        sh *= 2
    colcum = acc
    tot = colcum[R - 1:R, :]
    ia = lax.broadcasted_iota(jnp.int32, (C, C), 0)
    ib = lax.broadcasted_iota(jnp.int32, (C, C), 1)
    tri = (ia < ib).astype(jnp.float32)
    excl = jnp.dot(tot, tri, preferred_element_type=jnp.float32)
    cum_t = colcum + excl

    gts = jnp.sum(t_s)
    cnt = (idx + 1).astype(jnp.float32)
    cum1 = cnt - cum_t
    inter = gts - cum_t
    union = gts + cum1
    jacc = 1.0 - inter / jnp.maximum(union, 1e-6)
    wrapped = _roll(jacc, 1, 0)
    lastrow = jacc[R - 1:R, :]
    lastrow_sh = jnp.concatenate(
        [jnp.zeros((1, 1), jnp.float32), lastrow[:, :-1]], axis=1)
    prev = jnp.where(row == 0, lastrow_sh, wrapped)
    grad = jacc - prev
    loss = jnp.sum(jnp.maximum(e_s, 0.0) * grad)
    o_ref[0, 0, :] = jnp.broadcast_to(loss, (C,))


def _run(pred, target, interpret=False):
    B = pred.shape[0]
    C = 128
    R = (pred.shape[1] * pred.shape[2]) // C
    p = pred.reshape(B, R, C)
    t = target.reshape(B, R, C)
    losses = pl.pallas_call(
        _lovasz_body,
        grid=(B,),
        in_specs=[
            pl.BlockSpec((1, R, C), lambda i: (i, 0, 0)),
            pl.BlockSpec((1, R, C), lambda i: (i, 0, 0)),
        ],
        out_specs=pl.BlockSpec((1, 1, C), lambda i: (i, 0, 0)),
        out_shape=jax.ShapeDtypeStruct((B, 1, C), jnp.float32),
        interpret=interpret,
    )(p, t)
    total = jnp.sum(losses[:, 0, 0]) / B
    return jnp.where(jnp.isfinite(total), total, jnp.zeros((), jnp.float32))


def kernel(pred, target):
    return _run(pred, target)


# speedup vs baseline: 1.5333x; 1.1644x over previous
import jax
import jax.numpy as jnp
from jax import lax
from jax.experimental import pallas as pl


def _roll(x, sh, axis):
    if axis == 0:
        return jnp.concatenate([x[-sh:, :], x[:-sh, :]], axis=0)
    return jnp.concatenate([x[:, -sh:], x[:, :-sh]], axis=1)


def _roll3(x, sh, axis):
    idx = [slice(None)] * 3
    lo = [slice(None)] * 3
    idx[axis] = slice(-sh, None)
    lo[axis] = slice(None, -sh)
    return jnp.concatenate([x[tuple(idx)], x[tuple(lo)]], axis=axis)


def _lovasz_body(p_ref, t_ref, o_ref):
    p = p_ref[0]
    t = t_ref[0]
    R, C = p.shape
    N = R * C
    LOGN = N.bit_length() - 1

    row = lax.broadcasted_iota(jnp.int32, (R, C), 0)
    lane = lax.broadcasted_iota(jnp.int32, (R, C), 1)

    signs = 2.0 * t - 1.0
    e = 1.0 - p * signs
    bits = lax.bitcast_convert_type(e, jnp.int32)
    bits = (bits & jnp.int32(-2)) | t.astype(jnp.int32)
    y = bits ^ ((bits >> 31) & jnp.int32(0x7FFFFFFF))
    z = ~y

    R8 = R // 8
    a0 = lax.broadcasted_iota(jnp.int32, (R8, 8, C), 0)
    a1 = lax.broadcasted_iota(jnp.int32, (R8, 8, C), 1)
    ln3 = lax.broadcasted_iota(jnp.int32, (R8, 8, C), 2)
    idx3 = ln3 * R + a0 * 8 + a1

    def bit0(s):
        if s >= R:
            return (ln3 & (s // R)) == 0
        if s >= 8:
            return (a0 & (s // 8)) == 0
        return (a1 & s) == 0

    z = z.reshape(R8, 8, C)
    z = z ^ -((idx3 >> 1) & 1)
    for k in range(1, LOGN + 1):
        for j in range(k - 1, -1, -1):
            s = 1 << j
            bs = bit0(s)
            if s >= R:
                u, ax, wid = s // R, 2, C
            elif s >= 8:
                u, ax, wid = s // 8, 0, R8
            else:
                u, ax, wid = s, 1, 8
            w = _roll3(z, u, ax)
            m1 = jnp.minimum(z, w)
            m2 = jnp.maximum(z, w)
            mu = _roll3(m1, wid - u, ax)
            z = jnp.where(bs, mu, m2)
        if k < LOGN:
            z = z ^ -(((idx3 >> k) ^ (idx3 >> (k + 1))) & 1)
    z = z.reshape(R, C)

    y_s = ~z
    bits_s = y_s ^ ((y_s >> 31) & jnp.int32(0x7FFFFFFF))
    t_s = (bits_s & 1).astype(jnp.float32)
    e_s = lax.bitcast_convert_type(bits_s, jnp.float32)

    acc = t_s
    sh = 1
    while sh < R:
        shifted = jnp.concatenate(
            [jnp.zeros((sh, C), jnp.float32), acc[:-sh, :]], axis=0)
        acc = acc + shifted
        sh *= 2
    colcum = acc
    tot = colcum[R - 1:R, :]
    ia = lax.broadcasted_iota(jnp.int32, (C, C), 0)
    ib = lax.broadcasted_iota(jnp.int32, (C, C), 1)
    tri = (ia < ib).astype(jnp.float32)
    excl = jnp.dot(tot, tri, preferred_element_type=jnp.float32)
    cum_t = colcum + excl

    gts = jnp.sum(t_s)
    cnt = (lane * R + row + 1).astype(jnp.float32)
    cum1 = cnt - cum_t
    inter = gts - cum_t
    union = gts + cum1
    jacc = 1.0 - inter / jnp.maximum(union, 1e-6)
    wrapped = _roll(jacc, 1, 0)
    lastrow = jacc[R - 1:R, :]
    lastrow_sh = jnp.concatenate(
        [jnp.zeros((1, 1), jnp.float32), lastrow[:, :-1]], axis=1)
    prev = jnp.where(row == 0, lastrow_sh, wrapped)
    grad = jacc - prev
    loss = jnp.sum(jnp.maximum(e_s, 0.0) * grad)
    o_ref[0, 0, :] = jnp.broadcast_to(loss, (C,))


def _run(pred, target, interpret=False):
    B = pred.shape[0]
    C = 128
    R = (pred.shape[1] * pred.shape[2]) // C
    p = pred.reshape(B, R, C)
    t = target.reshape(B, R, C)
    losses = pl.pallas_call(
        _lovasz_body,
        grid=(B,),
        in_specs=[
            pl.BlockSpec((1, R, C), lambda i: (i, 0, 0)),
            pl.BlockSpec((1, R, C), lambda i: (i, 0, 0)),
        ],
        out_specs=pl.BlockSpec((1, 1, C), lambda i: (i, 0, 0)),
        out_shape=jax.ShapeDtypeStruct((B, 1, C), jnp.float32),
        interpret=interpret,
    )(p, t)
    total = jnp.sum(losses[:, 0, 0]) / B
    return jnp.where(jnp.isfinite(total), total, jnp.zeros((), jnp.float32))


def kernel(pred, target):
    return _run(pred, target)


# speedup vs baseline: 1.5514x; 1.0118x over previous
import jax
import jax.numpy as jnp
from jax import lax
from jax.experimental import pallas as pl


def _roll(x, sh, axis):
    if axis == 0:
        return jnp.concatenate([x[-sh:, :], x[:-sh, :]], axis=0)
    return jnp.concatenate([x[:, -sh:], x[:, :-sh]], axis=1)


def _roll3(x, sh, axis):
    idx = [slice(None)] * 3
    lo = [slice(None)] * 3
    idx[axis] = slice(-sh, None)
    lo[axis] = slice(None, -sh)
    return jnp.concatenate([x[tuple(idx)], x[tuple(lo)]], axis=axis)


def _lovasz_body(p_ref, t_ref, o_ref):
    p = p_ref[0]
    t = t_ref[0]
    R, C = p.shape
    N = R * C
    LOGN = N.bit_length() - 1

    row = lax.broadcasted_iota(jnp.int32, (R, C), 0)
    lane = lax.broadcasted_iota(jnp.int32, (R, C), 1)

    signs = 2.0 * t - 1.0
    e = 1.0 - p * signs
    bits = lax.bitcast_convert_type(e, jnp.int32)
    bits = (bits & jnp.int32(-2)) | t.astype(jnp.int32)
    y = bits ^ ((bits >> 31) & jnp.int32(0x7FFFFFFF))
    z = ~y

    R8 = R // 8
    a0 = lax.broadcasted_iota(jnp.int32, (R8, 8, C), 0)
    a1 = lax.broadcasted_iota(jnp.int32, (R8, 8, C), 1)
    ln3 = lax.broadcasted_iota(jnp.int32, (R8, 8, C), 2)
    idx3 = ln3 * R + a0 * 8 + a1

    def bit0(s):
        if s >= R:
            return (ln3 & (s // R)) == 0
        if s >= 8:
            return (a0 & (s // 8)) == 0
        return (a1 & s) == 0

    z = z.reshape(R8, 8, C)
    z = z ^ -((idx3 >> 1) & 1)
    for k in range(1, LOGN + 1):
        for j in range(k - 1, -1, -1):
            s = 1 << j
            bs = bit0(s)
            if s >= 8 and s < R:
                u = s // 8
                g = R8 // (2 * u)
                v = z.reshape(g, 2, u, 8, C)
                w = jnp.concatenate([v[:, 1:], v[:, :1]], axis=1)
                w = w.reshape(R8, 8, C)
            else:
                if s >= R:
                    u, ax, wid = s // R, 2, C
                else:
                    u, ax, wid = s, 1, 8
                w_dn = _roll3(z, wid - u, ax)
                w_up = _roll3(z, u, ax)
                z = jnp.where(bs, jnp.minimum(z, w_dn),
                              jnp.maximum(z, w_up))
                continue
            z = jnp.where(bs, jnp.minimum(z, w), jnp.maximum(z, w))
        if k < LOGN:
            z = z ^ -(((idx3 >> k) ^ (idx3 >> (k + 1))) & 1)
    z = z.reshape(R, C)

    y_s = ~z
    bits_s = y_s ^ ((y_s >> 31) & jnp.int32(0x7FFFFFFF))
    t_s = (bits_s & 1).astype(jnp.float32)
    e_s = lax.bitcast_convert_type(bits_s, jnp.float32)

    acc = t_s
    sh = 1
    while sh < R:
        shifted = jnp.concatenate(
            [jnp.zeros((sh, C), jnp.float32), acc[:-sh, :]], axis=0)
        acc = acc + shifted
        sh *= 2
    colcum = acc
    tot = colcum[R - 1:R, :]
    ia = lax.broadcasted_iota(jnp.int32, (C, C), 0)
    ib = lax.broadcasted_iota(jnp.int32, (C, C), 1)
    tri = (ia < ib).astype(jnp.float32)
    excl = jnp.dot(tot, tri, preferred_element_type=jnp.float32)
    cum_t = colcum + excl

    gts = jnp.sum(t_s)
    cnt = (lane * R + row + 1).astype(jnp.float32)
    cum1 = cnt - cum_t
    inter = gts - cum_t
    union = gts + cum1
    jacc = 1.0 - inter / jnp.maximum(union, 1e-6)
    wrapped = _roll(jacc, 1, 0)
    lastrow = jacc[R - 1:R, :]
    lastrow_sh = jnp.concatenate(
        [jnp.zeros((1, 1), jnp.float32), lastrow[:, :-1]], axis=1)
    prev = jnp.where(row == 0, lastrow_sh, wrapped)
    grad = jacc - prev
    loss = jnp.sum(jnp.maximum(e_s, 0.0) * grad)
    o_ref[0, 0, :] = jnp.broadcast_to(loss, (C,))


def _run(pred, target, interpret=False):
    B = pred.shape[0]
    C = 128
    R = (pred.shape[1] * pred.shape[2]) // C
    p = pred.reshape(B, R, C)
    t = target.reshape(B, R, C)
    losses = pl.pallas_call(
        _lovasz_body,
        grid=(B,),
        in_specs=[
            pl.BlockSpec((1, R, C), lambda i: (i, 0, 0)),
            pl.BlockSpec((1, R, C), lambda i: (i, 0, 0)),
        ],
        out_specs=pl.BlockSpec((1, 1, C), lambda i: (i, 0, 0)),
        out_shape=jax.ShapeDtypeStruct((B, 1, C), jnp.float32),
        interpret=interpret,
    )(p, t)
    total = jnp.sum(losses[:, 0, 0]) / B
    return jnp.where(jnp.isfinite(total), total, jnp.zeros((), jnp.float32))


def kernel(pred, target):
    return _run(pred, target)
